# baseline (speedup 1.0000x reference)
import functools

import numpy as np

import jax
import jax.numpy as jnp
from jax import lax
from jax.experimental import pallas as pl
from jax.experimental.pallas import tpu as pltpu
from jax.experimental.pallas import tpu_sc as plsc

_G = 512
_N = 4194304
_ROWS = 32768
_TBR = 512
_NC, _NS = 2, 16
_NW = _NC * _NS
_PW = _N // _NW
_WIN = 8192


def _tc1_body(x_ref, s_ref, t_ref, wx_ref, wlo_ref, widx_ref, aux_ref):
    f = jnp.round(x_ref[...] * s_ref[...] + t_ref[...])
    f = jnp.clip(f, -4.0, 516.0)
    hi = lax.dot(f, wx_ref[...], precision=lax.Precision.HIGHEST,
                 preferred_element_type=jnp.float32)
    lo = lax.dot(f, wlo_ref[...], precision=lax.Precision.HIGHEST,
                 preferred_element_type=jnp.float32)
    hii = hi.astype(jnp.int32)
    loi = lo.astype(jnp.int32)
    jj = loi >> 10
    kk = loi & 1023
    inb = ((hii >= 0) & (hii < _G) & (jj >= 0) & (jj < _G)
           & (kk >= 0) & (kk < _G))
    lin = ((jnp.clip(hii, 0, _G - 1) << 18)
           | (jnp.clip(jj, 0, _G - 1) << 9)
           | jnp.clip(kk, 0, _G - 1))
    widx_ref[...] = (lin >> 2).reshape(_TBR * 128)
    aux = ((lin & 3) << 3) | (inb.astype(jnp.int32) << 5)
    aux_ref[...] = aux.astype(jnp.int8).reshape(_TBR * 128)


_tc1 = pl.pallas_call(
    _tc1_body,
    grid=(_ROWS // _TBR,),
    in_specs=[
        pl.BlockSpec((_TBR, 384), lambda g: (g, 0)),
        pl.BlockSpec((1, 384), lambda g: (0, 0)),
        pl.BlockSpec((1, 384), lambda g: (0, 0)),
        pl.BlockSpec((384, 128), lambda g: (0, 0)),
        pl.BlockSpec((384, 128), lambda g: (0, 0)),
    ],
    out_specs=[
        pl.BlockSpec((_TBR * 128,), lambda g: (g,)),
        pl.BlockSpec((_TBR * 128,), lambda g: (g,)),
    ],
    out_shape=[
        jax.ShapeDtypeStruct((_N,), jnp.int32),
        jax.ShapeDtypeStruct((_N,), jnp.int8),
    ],
)


def _sc_body(tbl_hbm, widx_hbm, out_hbm, idx_v, gw_v, sem):
    wid = lax.axis_index("s") * _NC + lax.axis_index("c")
    base = wid * _PW

    def win_body(w, carry):
        off = base + w * _WIN
        pltpu.sync_copy(widx_hbm.at[pl.ds(off, _WIN)], idx_v)

        def issue(i, c):
            v = idx_v[pl.ds(i * 16, 16)]
            pltpu.async_copy(tbl_hbm.at[v], gw_v.at[pl.ds(i * 16, 16)], sem)
            return c

        lax.fori_loop(0, _WIN // 16, issue, 0, unroll=8)
        pltpu.make_async_copy(widx_hbm.at[pl.ds(off, _WIN)], gw_v, sem).wait()
        pltpu.sync_copy(gw_v, out_hbm.at[pl.ds(off, _WIN)])
        return carry

    lax.fori_loop(0, _PW // _WIN, win_body, 0)


@functools.cache
def _sc_gather_kernel():
    mesh = plsc.VectorSubcoreMesh(core_axis_name="c", subcore_axis_name="s",
                                  num_cores=_NC, num_subcores=_NS)
    return pl.kernel(
        _sc_body,
        out_type=jax.ShapeDtypeStruct((_N,), jnp.int32),
        mesh=mesh,
        scratch_types=[
            pltpu.VMEM((_WIN,), jnp.int32),
            pltpu.VMEM((_WIN,), jnp.int32),
            pltpu.SemaphoreType.DMA,
        ],
    )


def _tc2_body(gw_ref, aux_ref, o_ref):
    w = gw_ref[...]
    a = aux_ref[...].astype(jnp.int32)
    bit = (w >> (a & 31)) & (a >> 5)
    o_ref[...] = bit == 1


_tc2 = pl.pallas_call(
    _tc2_body,
    grid=(32,),
    in_specs=[
        pl.BlockSpec((_N // 32,), lambda g: (g,)),
        pl.BlockSpec((_N // 32,), lambda g: (g,)),
    ],
    out_specs=pl.BlockSpec((_N // 32,), lambda g: (g,)),
    out_shape=jax.ShapeDtypeStruct((_N,), jnp.bool_),
)


def _sel_weights():
    wx = np.zeros((384, 128), np.float32)
    wlo = np.zeros((384, 128), np.float32)
    for i in range(384):
        j = i // 3
        c = i % 3
        if c == 0:
            wx[i, j] = 1.0
        elif c == 1:
            wlo[i, j] = 1024.0
        else:
            wlo[i, j] = 1.0
    return jnp.asarray(wx), jnp.asarray(wlo)


def kernel(xyz, mask, xyz2ijk_scale, xyz2ijk_shift):
    xmat = xyz.reshape(_ROWS, 384)
    svec = jnp.tile(xyz2ijk_scale, 128).reshape(1, 384)
    tvec = jnp.tile(xyz2ijk_shift, 128).reshape(1, 384)
    wx, wlo = _sel_weights()
    tbl = lax.bitcast_convert_type(
        mask.astype(jnp.uint8).reshape(_G, _G, _G // 4, 4), jnp.int32)
    tbl = tbl.reshape(_G * _G * (_G // 4))
    widx, aux = _tc1(xmat, svec, tvec, wx, wlo)
    gw = _sc_gather_kernel()(tbl, widx)
    out = _tc2(gw, aux)
    return out
